# baseline (speedup 1.0000x reference)
import functools

import jax
import jax.numpy as jnp
from jax import lax
from jax.experimental import pallas as pl
from jax.experimental.pallas import tpu as pltpu
from jax.experimental.pallas import tpu_sc as plsc

B, N, E = 2, 4096, 32768
NODE, EDGE, COND, HID, L, RBF, MAXZ = 192, 64, 128, 256, 4, 32, 100
CUTOFF = 5.0

F32 = jnp.float32
BF16 = jnp.bfloat16

NSUB = 16
CH = 128
E_PER_SUB = E // NSUB
N_CHUNKS = E_PER_SUB // CH
G_CHUNKS = 8
N_GROUPS = N_CHUNKS // G_CHUNKS
N_PER_SUB = N // NSUB

def _silu(x):
    return x * jax.nn.sigmoid(x)


def _embed_body(z_ref, frac_ref, tab_ref, cw1_ref, cb1_ref, cw2_ref, cb2_ref,
                wh_ref, cond_ref, wc_ref, mb1_ref, h_ref, pre_ref):
    blk = z_ref.shape[0]
    z = jnp.clip(z_ref[...], 0, MAXZ)
    onehot = (z == lax.broadcasted_iota(jnp.int32, (blk, 128), 1)).astype(F32)
    h = jnp.dot(onehot, tab_ref[...], preferred_element_type=F32)
    frac = frac_ref[...]
    t = (frac[:, 0:1] * cw1_ref[0:1, :] + frac[:, 1:2] * cw1_ref[1:2, :]
         + frac[:, 2:3] * cw1_ref[2:3, :] + cb1_ref[...])
    h = h + jnp.dot(_silu(t), cw2_ref[...], preferred_element_type=F32) + cb2_ref[...]
    h_ref[...] = h
    cm = jnp.dot(cond_ref[...], wc_ref[...], preferred_element_type=F32) + mb1_ref[...]
    pre_ref[...] = (jnp.dot(h, wh_ref[...], preferred_element_type=F32) + cm).astype(BF16)


def _embed_call(z2, frac2, tab_pad, cw1, cb1, cw2, cb2, wh, cond2, wc, mb1):
    blk = 1024
    grid = (B * N // blk,)
    full = lambda shape: pl.BlockSpec(shape, lambda i: (0, 0))
    return pl.pallas_call(
        _embed_body,
        grid=grid,
        in_specs=[
            pl.BlockSpec((blk, 1), lambda i: (i, 0)),
            pl.BlockSpec((blk, 3), lambda i: (i, 0)),
            full((128, NODE)), full((3, NODE)), full((1, NODE)),
            full((NODE, NODE)), full((1, NODE)),
            full((NODE, HID)), full((1, COND)), full((COND, HID)), full((1, HID)),
        ],
        out_specs=[pl.BlockSpec((blk, NODE), lambda i: (i, 0)),
                   pl.BlockSpec((blk, HID), lambda i: (i, 0))],
        out_shape=[jax.ShapeDtypeStruct((B * N, NODE), F32),
                   jax.ShapeDtypeStruct((B * N, HID), BF16)],
    )(z2, frac2, tab_pad, cw1, cb1, cw2, cb2, wh, cond2, wc, mb1)


def _edgefeat_body(dist_ref, gamma_ref, ew1_ref, eb1_ref, ew2_ref, eb2_ref, e_ref):
    blk = dist_ref.shape[0]
    d = jnp.clip(dist_ref[...], 0.0, CUTOFF)
    centers = (lax.broadcasted_iota(jnp.int32, (blk, RBF), 1).astype(F32)
               * (CUTOFF / (RBF - 1)))
    rbf = jnp.exp(-gamma_ref[0, 0] * (d - centers) ** 2)
    pre = (jnp.dot(rbf, ew1_ref[0:RBF, :], preferred_element_type=F32)
           + (d / CUTOFF) * ew1_ref[RBF:RBF + 1, :] + eb1_ref[...])
    e_ref[...] = jnp.dot(_silu(pre), ew2_ref[...], preferred_element_type=F32) + eb2_ref[...]


def _edgefeat_call(dist2, gamma11, ew1, eb1, ew2, eb2):
    blk = 2048
    grid = (B * E // blk,)
    full = lambda shape: pl.BlockSpec(shape, lambda i: (0, 0))
    return pl.pallas_call(
        _edgefeat_body,
        grid=grid,
        in_specs=[
            pl.BlockSpec((blk, 1), lambda i: (i, 0)),
            full((1, 1)), full((RBF + 1, EDGE)), full((1, EDGE)),
            full((EDGE, EDGE)), full((1, EDGE)),
        ],
        out_specs=pl.BlockSpec((blk, EDGE), lambda i: (i, 0)),
        out_shape=jax.ShapeDtypeStruct((B * E, EDGE), F32),
    )(dist2, gamma11, ew1, eb1, ew2, eb2)


def _edgemlp_body(g_ref, e_ref, em_ref, we_ref, w2_ref, b2_ref, msg_ref):
    t = _silu(g_ref[...].astype(F32)
              + jnp.dot(e_ref[...], we_ref[...], preferred_element_type=F32))
    m = _silu(jnp.dot(t, w2_ref[...], preferred_element_type=F32) + b2_ref[...])
    msg_ref[...] = m * em_ref[...]


def _edgemlp_call(g2, e2, em2, we, w2, b2):
    blk = 2048
    grid = (B * E // blk,)
    full = lambda shape: pl.BlockSpec(shape, lambda i: (0, 0))
    return pl.pallas_call(
        _edgemlp_body,
        grid=grid,
        in_specs=[
            pl.BlockSpec((blk, HID), lambda i: (i, 0)),
            pl.BlockSpec((blk, EDGE), lambda i: (i, 0)),
            pl.BlockSpec((blk, 1), lambda i: (i, 0)),
            full((EDGE, HID)), full((HID, HID)), full((1, HID)),
        ],
        out_specs=pl.BlockSpec((blk, HID), lambda i: (i, 0)),
        out_shape=jax.ShapeDtypeStruct((B * E, HID), F32),
    )(g2, e2, em2, we, w2, b2)


def _node_body(with_pre, h_ref, agg_ref, mf_ref, cond_ref, uh_ref, ua_ref,
               uc_ref, ub1_ref, u2_ref, ub2_ref, lng_ref, lnb_ref,
               whn_ref, wcn_ref, mb1n_ref, hn_ref, pre_ref):
    h = h_ref[...]
    cu = jnp.dot(cond_ref[...], uc_ref[...], preferred_element_type=F32) + ub1_ref[...]
    u1 = _silu(jnp.dot(h, uh_ref[...], preferred_element_type=F32)
               + jnp.dot(agg_ref[...], ua_ref[...], preferred_element_type=F32) + cu)
    dh = jnp.dot(u1, u2_ref[...], preferred_element_type=F32) + ub2_ref[...]
    x = h + dh
    mu = jnp.mean(x, axis=-1, keepdims=True)
    xc = x - mu
    var = jnp.mean(xc * xc, axis=-1, keepdims=True)
    out = xc * lax.rsqrt(var + 1e-5) * lng_ref[...] + lnb_ref[...]
    mf = mf_ref[...]
    hn = mf * out + (1.0 - mf) * h
    hn_ref[...] = hn
    if with_pre:
        cm = jnp.dot(cond_ref[...], wcn_ref[...], preferred_element_type=F32) + mb1n_ref[...]
        pre_ref[...] = (jnp.dot(hn, whn_ref[...], preferred_element_type=F32) + cm).astype(BF16)


def _node_call(with_pre, h2, agg2, mf2, cond2, uh, ua, uc, ub1, u2, ub2,
               lng, lnb, whn, wcn, mb1n):
    blk = 1024
    grid = (B * N // blk,)
    full = lambda shape: pl.BlockSpec(shape, lambda i: (0, 0))
    out_specs = [pl.BlockSpec((blk, NODE), lambda i: (i, 0))]
    out_shape = [jax.ShapeDtypeStruct((B * N, NODE), F32)]
    if with_pre:
        out_specs.append(pl.BlockSpec((blk, HID), lambda i: (i, 0)))
        out_shape.append(jax.ShapeDtypeStruct((B * N, HID), BF16))
    body = functools.partial(_node_body, with_pre)
    if not with_pre:
        def body(h_ref, agg_ref, mf_ref, cond_ref, uh_ref, ua_ref, uc_ref,
                 ub1_ref, u2_ref, ub2_ref, lng_ref, lnb_ref, whn_ref, wcn_ref,
                 mb1n_ref, hn_ref):
            _node_body(False, h_ref, agg_ref, mf_ref, cond_ref, uh_ref, ua_ref,
                       uc_ref, ub1_ref, u2_ref, ub2_ref, lng_ref, lnb_ref,
                       whn_ref, wcn_ref, mb1n_ref, hn_ref, None)
    return pl.pallas_call(
        body,
        grid=grid,
        in_specs=[
            pl.BlockSpec((blk, NODE), lambda i: (i, 0)),
            pl.BlockSpec((blk, HID), lambda i: (i, 0)),
            pl.BlockSpec((blk, 1), lambda i: (i, 0)),
            full((1, COND)), full((NODE, HID)), full((HID, HID)),
            full((COND, HID)), full((1, HID)), full((HID, NODE)), full((1, NODE)),
            full((1, NODE)), full((1, NODE)),
            full((NODE, HID)), full((COND, HID)), full((1, HID)),
        ],
        out_specs=out_specs,
        out_shape=out_shape,
    )(h2, agg2, mf2, cond2, uh, ua, uc, ub1, u2, ub2, lng, lnb, whn, wcn, mb1n)


def _final_body(h_ref, mf_ref, pw1_ref, pb1_ref, pw2_ref, pb2_ref,
                hw_ref, hb_ref, out_ref):
    rows = []
    for b in range(B):
        h = h_ref[b]
        mf = mf_ref[b]
        denom = jnp.maximum(jnp.sum(mf, axis=0, keepdims=True), 1.0)
        rows.append(jnp.sum(h * mf, axis=0, keepdims=True) / denom)
    pooled = jnp.concatenate(rows, axis=0)
    f1 = _silu(jnp.dot(pooled, pw1_ref[...], preferred_element_type=F32) + pb1_ref[...])
    f2 = _silu(jnp.dot(f1, pw2_ref[...], preferred_element_type=F32) + pb2_ref[...])
    o = jnp.dot(f2, hw_ref[...], preferred_element_type=F32) + hb_ref[...]
    lanes = lax.broadcasted_iota(jnp.int32, (B, 3), 1)
    out_ref[...] = jnp.where(lanes == 2, jax.nn.sigmoid(o), o)


def _final_call(h3, mf3, pw1, pb1, pw2, pb2, hw, hb):
    full = lambda shape: pl.BlockSpec(shape, lambda: tuple(0 for _ in shape))
    return pl.pallas_call(
        _final_body,
        in_specs=[
            full((B, N, NODE)),
            full((B, N, 1)),
            full((NODE, HID)), full((1, HID)), full((HID, HID)), full((1, HID)),
            full((HID, 3)), full((1, 3)),
        ],
        out_specs=full((B, 3)),
        out_shape=jax.ShapeDtypeStruct((B, 3), F32),
    )(h3, mf3, pw1, pb1, pw2, pb2, hw, hb)


@functools.cache
def _sc_gather_kernel():
    mesh = plsc.VectorSubcoreMesh(core_axis_name="c", subcore_axis_name="s")

    @functools.partial(
        pl.kernel, mesh=mesh,
        out_type=jax.ShapeDtypeStruct((B, E, HID // 2), jnp.int32),
        scratch_types=[
            pltpu.VMEM((E_PER_SUB,), jnp.int32),
            pltpu.VMEM((2, CH, HID // 2), jnp.int32),
            pltpu.SemaphoreType.DMA,
            pltpu.SemaphoreType.DMA,
            pltpu.SemaphoreType.DMA,
            pltpu.SemaphoreType.DMA,
        ],
    )
    def gk(tab_hbm, idx_hbm, out_hbm, idx_v, buf, g0, g1, o0, o1):
        c = lax.axis_index("c")
        s = lax.axis_index("s")
        base = s * E_PER_SUB
        pltpu.sync_copy(idx_hbm.at[pl.ds(c * E + base, E_PER_SUB)], idx_v)
        gsem = (g0, g1)
        osem = (o0, o1)

        def group(g, carry):
            j0 = g * G_CHUNKS
            gh = [None, None]
            oh = [None, None]

            def start_gather(t):
                b = t & 1
                gh[b] = pltpu.async_copy(
                    tab_hbm.at[idx_v.at[pl.ds((j0 + t) * CH, CH)]],
                    buf.at[b], gsem[b])

            start_gather(0)
            for t in range(G_CHUNKS):
                b = t & 1
                nb = b ^ 1
                if t + 1 < G_CHUNKS:
                    if t >= 1:
                        oh[nb].wait()
                    start_gather(t + 1)
                gh[b].wait()
                oh[b] = pltpu.async_copy(
                    buf.at[b],
                    out_hbm.at[c, pl.ds(base + (j0 + t) * CH, CH)], osem[b])
            oh[0].wait()
            oh[1].wait()
            return carry

        lax.fori_loop(0, N_GROUPS, group, 0)

    return gk


def _sc_gather(pre_flat, src2):
    pre_i = lax.bitcast_convert_type(
        pre_flat.reshape(B * N, HID // 2, 2), jnp.int32)
    g_i = _sc_gather_kernel()(pre_i, src2)
    return lax.bitcast_convert_type(g_i, BF16).reshape(B, E, HID)


@functools.cache
def _sc_scatter_kernel():
    mesh = plsc.VectorSubcoreMesh(core_axis_name="c", subcore_axis_name="s")

    @functools.partial(
        pl.kernel, mesh=mesh,
        out_type=jax.ShapeDtypeStruct((B * N, HID), F32),
        scratch_types=[
            pltpu.VMEM((E_PER_SUB,), jnp.int32),
            pltpu.VMEM((2, CH, HID), F32),
            pltpu.SemaphoreType.DMA,
            pltpu.SemaphoreType.DMA,
            pltpu.SemaphoreType.DMA,
            pltpu.SemaphoreType.DMA,
        ],
    )
    def sk(msg_hbm, dst_hbm, zer_hbm, out_hbm, idx_v, buf, m0, m1, a0, a1):
        c = lax.axis_index("c")
        s = lax.axis_index("s")
        base = s * E_PER_SUB
        pltpu.sync_copy(zer_hbm.at[pl.ds(s * N_PER_SUB, N_PER_SUB)],
                        out_hbm.at[pl.ds(c * N + s * N_PER_SUB, N_PER_SUB)])
        pltpu.sync_copy(dst_hbm.at[pl.ds(c * E + base, E_PER_SUB)], idx_v)
        plsc.subcore_barrier()
        msem = (m0, m1)
        asem = (a0, a1)

        def group(g, carry):
            j0 = g * G_CHUNKS
            mh = [None, None]
            ah = [None, None]

            def start_load(t):
                b = t & 1
                mh[b] = pltpu.async_copy(
                    msg_hbm.at[pl.ds(c * E + base + (j0 + t) * CH, CH)],
                    buf.at[b], msem[b])

            start_load(0)
            for t in range(G_CHUNKS):
                b = t & 1
                nb = b ^ 1
                if t + 1 < G_CHUNKS:
                    if t >= 1:
                        ah[nb].wait()
                    start_load(t + 1)
                mh[b].wait()
                ah[b] = pltpu.async_copy(
                    buf.at[b],
                    out_hbm.at[idx_v.at[pl.ds((j0 + t) * CH, CH)]],
                    asem[b], add=True)
            ah[0].wait()
            ah[1].wait()
            return carry

        lax.fori_loop(0, N_GROUPS, group, 0)

    return sk


def _sc_scatter(msg2, dst2, zer):
    return _sc_scatter_kernel()(msg2, dst2, zer)


def kernel(z, frac_coords, edge_index, dist, mask, edge_mask, atom_table,
           coord_W1, coord_b1, coord_W2, coord_b2, rbf_gamma, edge_W1, edge_b1,
           edge_W2, edge_b2, null_cond, msg_W1, msg_b1, msg_W2, msg_b2,
           upd_W1, upd_b1, upd_W2, upd_b2, ln_g, ln_b, pool_W1, pool_b1,
           pool_W2, pool_b2, head_W, head_b):
    z2 = z.reshape(B * N, 1).astype(jnp.int32)
    frac2 = frac_coords.reshape(B * N, 3).astype(F32)
    src = edge_index[0].astype(jnp.int32)
    dst = edge_index[1].astype(jnp.int32)
    src2 = jnp.concatenate([src + b * N for b in range(B)])
    dst2 = jnp.concatenate([dst + b * N for b in range(B)])
    mf2 = mask.astype(F32).reshape(B * N, 1)
    mf3 = mask.astype(F32).reshape(B, N, 1)
    em2 = edge_mask.astype(F32).reshape(B * E, 1)
    dist2 = dist.reshape(B * E, 1).astype(F32)
    gamma11 = rbf_gamma.reshape(1, 1).astype(F32)
    cond2 = null_cond.reshape(1, COND)
    tab_pad = jnp.pad(atom_table, ((0, 128 - (MAXZ + 1)), (0, 0)))
    zer = jnp.zeros((N, HID), F32)

    r1 = lambda v: v.reshape(1, -1)

    h2, pre = _embed_call(z2, frac2, tab_pad, coord_W1, r1(coord_b1),
                          coord_W2, r1(coord_b2), msg_W1[0][:NODE], cond2,
                          msg_W1[0][NODE + EDGE:], r1(msg_b1[0]))
    e2 = _edgefeat_call(dist2, gamma11, edge_W1, r1(edge_b1), edge_W2, r1(edge_b2))

    for i in range(L):
        g = _sc_gather(pre, src2)
        msg2 = _edgemlp_call(g.reshape(B * E, HID), e2, em2,
                             msg_W1[i][NODE:NODE + EDGE], msg_W2[i], r1(msg_b2[i]))
        agg = _sc_scatter(msg2, dst2, zer)
        nxt = i + 1
        with_pre = nxt < L
        wi = nxt if with_pre else i
        outs = _node_call(with_pre, h2, agg, mf2, cond2,
                          upd_W1[i][:NODE], upd_W1[i][NODE:NODE + HID],
                          upd_W1[i][NODE + HID:], r1(upd_b1[i]), upd_W2[i],
                          r1(upd_b2[i]), r1(ln_g[i]), r1(ln_b[i]),
                          msg_W1[wi][:NODE], msg_W1[wi][NODE + EDGE:], r1(msg_b1[wi]))
        if with_pre:
            h2, pre = outs
        else:
            (h2,) = outs

    return _final_call(h2.reshape(B, N, NODE), mf3, pool_W1, r1(pool_b1),
                       pool_W2, r1(pool_b2), head_W, r1(head_b))


# speedup vs baseline: 1.5037x; 1.5037x over previous
import functools

import jax
import jax.numpy as jnp
from jax import lax
from jax.experimental import pallas as pl
from jax.experimental.pallas import tpu as pltpu
from jax.experimental.pallas import tpu_sc as plsc

B, N, E = 2, 4096, 32768
NODE, EDGE, COND, HID, L, RBF, MAXZ = 192, 64, 128, 256, 4, 32, 100
CUTOFF = 5.0

F32 = jnp.float32
BF16 = jnp.bfloat16

NSUB = 16
CH = 128
E_PER_SUB = E // NSUB
N_CHUNKS = E_PER_SUB // CH
G_CHUNKS = 8
N_GROUPS = N_CHUNKS // G_CHUNKS
N_PER_SUB = N // NSUB
GCH = 64
G_N_CHUNKS = E_PER_SUB // GCH
G_G_CHUNKS = 8
G_N_GROUPS = G_N_CHUNKS // G_G_CHUNKS

def _silu(x):
    return x * jax.nn.sigmoid(x)


def _embed_body(z_ref, frac_ref, tab_ref, cw1_ref, cb1_ref, cw2_ref, cb2_ref,
                wh_ref, cond_ref, wc_ref, mb1_ref, h_ref, pre_ref):
    blk = z_ref.shape[0]
    z = jnp.clip(z_ref[...], 0, MAXZ)
    onehot = (z == lax.broadcasted_iota(jnp.int32, (blk, 128), 1)).astype(F32)
    h = jnp.dot(onehot, tab_ref[...], preferred_element_type=F32)
    frac = frac_ref[...]
    t = (frac[:, 0:1] * cw1_ref[0:1, :] + frac[:, 1:2] * cw1_ref[1:2, :]
         + frac[:, 2:3] * cw1_ref[2:3, :] + cb1_ref[...])
    h = h + jnp.dot(_silu(t), cw2_ref[...], preferred_element_type=F32) + cb2_ref[...]
    h_ref[...] = h
    cm = jnp.dot(cond_ref[...], wc_ref[...], preferred_element_type=F32) + mb1_ref[...]
    pre_ref[...] = jnp.dot(h, wh_ref[...], preferred_element_type=F32) + cm


def _embed_call(z2, frac2, tab_pad, cw1, cb1, cw2, cb2, wh, cond2, wc, mb1):
    blk = 1024
    grid = (B * N // blk,)
    full = lambda shape: pl.BlockSpec(shape, lambda i: (0, 0))
    return pl.pallas_call(
        _embed_body,
        grid=grid,
        in_specs=[
            pl.BlockSpec((blk, 1), lambda i: (i, 0)),
            pl.BlockSpec((blk, 3), lambda i: (i, 0)),
            full((128, NODE)), full((3, NODE)), full((1, NODE)),
            full((NODE, NODE)), full((1, NODE)),
            full((NODE, HID)), full((1, COND)), full((COND, HID)), full((1, HID)),
        ],
        out_specs=[pl.BlockSpec((blk, NODE), lambda i: (i, 0)),
                   pl.BlockSpec((blk, HID), lambda i: (i, 0))],
        out_shape=[jax.ShapeDtypeStruct((B * N, NODE), F32),
                   jax.ShapeDtypeStruct((B * N, HID), F32)],
    )(z2, frac2, tab_pad, cw1, cb1, cw2, cb2, wh, cond2, wc, mb1)


def _edgefeat_body(dist_ref, gamma_ref, ew1_ref, eb1_ref, ew2_ref, eb2_ref, e_ref):
    blk = dist_ref.shape[0]
    d = jnp.clip(dist_ref[...], 0.0, CUTOFF)
    centers = (lax.broadcasted_iota(jnp.int32, (blk, RBF), 1).astype(F32)
               * (CUTOFF / (RBF - 1)))
    rbf = jnp.exp(-gamma_ref[0, 0] * (d - centers) ** 2)
    pre = (jnp.dot(rbf, ew1_ref[0:RBF, :], preferred_element_type=F32)
           + (d / CUTOFF) * ew1_ref[RBF:RBF + 1, :] + eb1_ref[...])
    e_ref[...] = jnp.dot(_silu(pre), ew2_ref[...], preferred_element_type=F32) + eb2_ref[...]


def _edgefeat_call(dist2, gamma11, ew1, eb1, ew2, eb2):
    blk = 2048
    grid = (B * E // blk,)
    full = lambda shape: pl.BlockSpec(shape, lambda i: (0, 0))
    return pl.pallas_call(
        _edgefeat_body,
        grid=grid,
        in_specs=[
            pl.BlockSpec((blk, 1), lambda i: (i, 0)),
            full((1, 1)), full((RBF + 1, EDGE)), full((1, EDGE)),
            full((EDGE, EDGE)), full((1, EDGE)),
        ],
        out_specs=pl.BlockSpec((blk, EDGE), lambda i: (i, 0)),
        out_shape=jax.ShapeDtypeStruct((B * E, EDGE), F32),
    )(dist2, gamma11, ew1, eb1, ew2, eb2)


def _edgemlp_body(g_ref, e_ref, em_ref, we_ref, w2_ref, b2_ref, msg_ref):
    t = _silu(g_ref[...].astype(F32)
              + jnp.dot(e_ref[...], we_ref[...], preferred_element_type=F32))
    m = _silu(jnp.dot(t, w2_ref[...], preferred_element_type=F32) + b2_ref[...])
    msg_ref[...] = m * em_ref[...]


def _edgemlp_call(g2, e2, em2, we, w2, b2):
    blk = 2048
    grid = (B * E // blk,)
    full = lambda shape: pl.BlockSpec(shape, lambda i: (0, 0))
    return pl.pallas_call(
        _edgemlp_body,
        grid=grid,
        in_specs=[
            pl.BlockSpec((blk, HID), lambda i: (i, 0)),
            pl.BlockSpec((blk, EDGE), lambda i: (i, 0)),
            pl.BlockSpec((blk, 1), lambda i: (i, 0)),
            full((EDGE, HID)), full((HID, HID)), full((1, HID)),
        ],
        out_specs=pl.BlockSpec((blk, HID), lambda i: (i, 0)),
        out_shape=jax.ShapeDtypeStruct((B * E, HID), F32),
    )(g2, e2, em2, we, w2, b2)


def _node_body(with_pre, h_ref, agg_ref, mf_ref, cond_ref, uh_ref, ua_ref,
               uc_ref, ub1_ref, u2_ref, ub2_ref, lng_ref, lnb_ref,
               whn_ref, wcn_ref, mb1n_ref, hn_ref, pre_ref):
    h = h_ref[...]
    cu = jnp.dot(cond_ref[...], uc_ref[...], preferred_element_type=F32) + ub1_ref[...]
    u1 = _silu(jnp.dot(h, uh_ref[...], preferred_element_type=F32)
               + jnp.dot(agg_ref[...], ua_ref[...], preferred_element_type=F32) + cu)
    dh = jnp.dot(u1, u2_ref[...], preferred_element_type=F32) + ub2_ref[...]
    x = h + dh
    mu = jnp.mean(x, axis=-1, keepdims=True)
    xc = x - mu
    var = jnp.mean(xc * xc, axis=-1, keepdims=True)
    out = xc * lax.rsqrt(var + 1e-5) * lng_ref[...] + lnb_ref[...]
    mf = mf_ref[...]
    hn = mf * out + (1.0 - mf) * h
    hn_ref[...] = hn
    if with_pre:
        cm = jnp.dot(cond_ref[...], wcn_ref[...], preferred_element_type=F32) + mb1n_ref[...]
        pre_ref[...] = jnp.dot(hn, whn_ref[...], preferred_element_type=F32) + cm


def _node_call(with_pre, h2, agg2, mf2, cond2, uh, ua, uc, ub1, u2, ub2,
               lng, lnb, whn, wcn, mb1n):
    blk = 1024
    grid = (B * N // blk,)
    full = lambda shape: pl.BlockSpec(shape, lambda i: (0, 0))
    out_specs = [pl.BlockSpec((blk, NODE), lambda i: (i, 0))]
    out_shape = [jax.ShapeDtypeStruct((B * N, NODE), F32)]
    if with_pre:
        out_specs.append(pl.BlockSpec((blk, HID), lambda i: (i, 0)))
        out_shape.append(jax.ShapeDtypeStruct((B * N, HID), F32))
    body = functools.partial(_node_body, with_pre)
    if not with_pre:
        def body(h_ref, agg_ref, mf_ref, cond_ref, uh_ref, ua_ref, uc_ref,
                 ub1_ref, u2_ref, ub2_ref, lng_ref, lnb_ref, whn_ref, wcn_ref,
                 mb1n_ref, hn_ref):
            _node_body(False, h_ref, agg_ref, mf_ref, cond_ref, uh_ref, ua_ref,
                       uc_ref, ub1_ref, u2_ref, ub2_ref, lng_ref, lnb_ref,
                       whn_ref, wcn_ref, mb1n_ref, hn_ref, None)
    return pl.pallas_call(
        body,
        grid=grid,
        in_specs=[
            pl.BlockSpec((blk, NODE), lambda i: (i, 0)),
            pl.BlockSpec((blk, HID), lambda i: (i, 0)),
            pl.BlockSpec((blk, 1), lambda i: (i, 0)),
            full((1, COND)), full((NODE, HID)), full((HID, HID)),
            full((COND, HID)), full((1, HID)), full((HID, NODE)), full((1, NODE)),
            full((1, NODE)), full((1, NODE)),
            full((NODE, HID)), full((COND, HID)), full((1, HID)),
        ],
        out_specs=out_specs,
        out_shape=out_shape,
    )(h2, agg2, mf2, cond2, uh, ua, uc, ub1, u2, ub2, lng, lnb, whn, wcn, mb1n)


def _final_body(h_ref, mf_ref, pw1_ref, pb1_ref, pw2_ref, pb2_ref,
                hw_ref, hb_ref, out_ref):
    rows = []
    for b in range(B):
        h = h_ref[b]
        mf = mf_ref[b]
        denom = jnp.maximum(jnp.sum(mf, axis=0, keepdims=True), 1.0)
        rows.append(jnp.sum(h * mf, axis=0, keepdims=True) / denom)
    pooled = jnp.concatenate(rows, axis=0)
    f1 = _silu(jnp.dot(pooled, pw1_ref[...], preferred_element_type=F32) + pb1_ref[...])
    f2 = _silu(jnp.dot(f1, pw2_ref[...], preferred_element_type=F32) + pb2_ref[...])
    o = jnp.dot(f2, hw_ref[...], preferred_element_type=F32) + hb_ref[...]
    lanes = lax.broadcasted_iota(jnp.int32, (B, 3), 1)
    out_ref[...] = jnp.where(lanes == 2, jax.nn.sigmoid(o), o)


def _final_call(h3, mf3, pw1, pb1, pw2, pb2, hw, hb):
    full = lambda shape: pl.BlockSpec(shape, lambda: tuple(0 for _ in shape))
    return pl.pallas_call(
        _final_body,
        in_specs=[
            full((B, N, NODE)),
            full((B, N, 1)),
            full((NODE, HID)), full((1, HID)), full((HID, HID)), full((1, HID)),
            full((HID, 3)), full((1, 3)),
        ],
        out_specs=full((B, 3)),
        out_shape=jax.ShapeDtypeStruct((B, 3), F32),
    )(h3, mf3, pw1, pb1, pw2, pb2, hw, hb)


@functools.cache
def _sc_gather_kernel():
    mesh = plsc.VectorSubcoreMesh(core_axis_name="c", subcore_axis_name="s")

    @functools.partial(
        pl.kernel, mesh=mesh,
        out_type=jax.ShapeDtypeStruct((B, E, HID), F32),
        scratch_types=[
            pltpu.VMEM((E_PER_SUB,), jnp.int32),
            pltpu.VMEM((4, GCH, HID), F32),
            pltpu.SemaphoreType.DMA,
            pltpu.SemaphoreType.DMA,
            pltpu.SemaphoreType.DMA,
            pltpu.SemaphoreType.DMA,
            pltpu.SemaphoreType.DMA,
            pltpu.SemaphoreType.DMA,
            pltpu.SemaphoreType.DMA,
            pltpu.SemaphoreType.DMA,
        ],
    )
    def gk(tab_hbm, idx_hbm, out_hbm, idx_v, buf,
           g0, g1, g2, g3, o0, o1, o2, o3):
        c = lax.axis_index("c")
        s = lax.axis_index("s")
        base = s * E_PER_SUB
        pltpu.sync_copy(idx_hbm.at[pl.ds(c * E + base, E_PER_SUB)], idx_v)
        gsem = (g0, g1, g2, g3)
        osem = (o0, o1, o2, o3)

        def group(g, carry):
            j0 = g * G_G_CHUNKS
            gh = [None] * 4
            oh = [None] * 4

            def start_gather(t):
                b = t & 3
                gh[b] = pltpu.async_copy(
                    tab_hbm.at[idx_v.at[pl.ds((j0 + t) * GCH, GCH)]],
                    buf.at[b], gsem[b])

            for t in range(3):
                start_gather(t)
            for t in range(G_G_CHUNKS):
                b = t & 3
                if t + 3 < G_G_CHUNKS:
                    if t >= 1:
                        oh[(t + 3) & 3].wait()
                    start_gather(t + 3)
                gh[b].wait()
                oh[b] = pltpu.async_copy(
                    buf.at[b],
                    out_hbm.at[c, pl.ds(base + (j0 + t) * GCH, GCH)], osem[b])
            for b in range(4):
                oh[b].wait()
            return carry

        lax.fori_loop(0, G_N_GROUPS, group, 0)

    return gk


def _sc_gather(pre_flat, src2):
    return _sc_gather_kernel()(pre_flat, src2)


@functools.cache
def _sc_scatter_kernel():
    mesh = plsc.VectorSubcoreMesh(core_axis_name="c", subcore_axis_name="s")

    @functools.partial(
        pl.kernel, mesh=mesh,
        out_type=jax.ShapeDtypeStruct((B * N, HID), F32),
        scratch_types=[
            pltpu.VMEM((E_PER_SUB,), jnp.int32),
            pltpu.VMEM((2, CH, HID), F32),
            pltpu.SemaphoreType.DMA,
            pltpu.SemaphoreType.DMA,
            pltpu.SemaphoreType.DMA,
            pltpu.SemaphoreType.DMA,
        ],
    )
    def sk(msg_hbm, dst_hbm, zer_hbm, out_hbm, idx_v, buf, m0, m1, a0, a1):
        c = lax.axis_index("c")
        s = lax.axis_index("s")
        base = s * E_PER_SUB
        pltpu.sync_copy(zer_hbm.at[pl.ds(s * N_PER_SUB, N_PER_SUB)],
                        out_hbm.at[pl.ds(c * N + s * N_PER_SUB, N_PER_SUB)])
        pltpu.sync_copy(dst_hbm.at[pl.ds(c * E + base, E_PER_SUB)], idx_v)
        plsc.subcore_barrier()
        msem = (m0, m1)
        asem = (a0, a1)

        def group(g, carry):
            j0 = g * G_CHUNKS
            mh = [None, None]
            ah = [None, None]

            def start_load(t):
                b = t & 1
                mh[b] = pltpu.async_copy(
                    msg_hbm.at[pl.ds(c * E + base + (j0 + t) * CH, CH)],
                    buf.at[b], msem[b])

            start_load(0)
            for t in range(G_CHUNKS):
                b = t & 1
                nb = b ^ 1
                if t + 1 < G_CHUNKS:
                    if t >= 1:
                        ah[nb].wait()
                    start_load(t + 1)
                mh[b].wait()
                ah[b] = pltpu.async_copy(
                    buf.at[b],
                    out_hbm.at[idx_v.at[pl.ds((j0 + t) * CH, CH)]],
                    asem[b], add=True)
            ah[0].wait()
            ah[1].wait()
            return carry

        lax.fori_loop(0, N_GROUPS, group, 0)

    return sk


def _sc_scatter(msg2, dst2, zer):
    return _sc_scatter_kernel()(msg2, dst2, zer)


def kernel(z, frac_coords, edge_index, dist, mask, edge_mask, atom_table,
           coord_W1, coord_b1, coord_W2, coord_b2, rbf_gamma, edge_W1, edge_b1,
           edge_W2, edge_b2, null_cond, msg_W1, msg_b1, msg_W2, msg_b2,
           upd_W1, upd_b1, upd_W2, upd_b2, ln_g, ln_b, pool_W1, pool_b1,
           pool_W2, pool_b2, head_W, head_b):
    z2 = z.reshape(B * N, 1).astype(jnp.int32)
    frac2 = frac_coords.reshape(B * N, 3).astype(F32)
    src = edge_index[0].astype(jnp.int32)
    dst = edge_index[1].astype(jnp.int32)
    src2 = jnp.concatenate([src + b * N for b in range(B)])
    dst2 = jnp.concatenate([dst + b * N for b in range(B)])
    mf2 = mask.astype(F32).reshape(B * N, 1)
    mf3 = mask.astype(F32).reshape(B, N, 1)
    em2 = edge_mask.astype(F32).reshape(B * E, 1)
    dist2 = dist.reshape(B * E, 1).astype(F32)
    gamma11 = rbf_gamma.reshape(1, 1).astype(F32)
    cond2 = null_cond.reshape(1, COND)
    tab_pad = jnp.pad(atom_table, ((0, 128 - (MAXZ + 1)), (0, 0)))
    zer = jnp.zeros((N, HID), F32)

    r1 = lambda v: v.reshape(1, -1)

    h2, pre = _embed_call(z2, frac2, tab_pad, coord_W1, r1(coord_b1),
                          coord_W2, r1(coord_b2), msg_W1[0][:NODE], cond2,
                          msg_W1[0][NODE + EDGE:], r1(msg_b1[0]))
    e2 = _edgefeat_call(dist2, gamma11, edge_W1, r1(edge_b1), edge_W2, r1(edge_b2))

    for i in range(L):
        g = _sc_gather(pre, src2)
        msg2 = _edgemlp_call(g.reshape(B * E, HID), e2, em2,
                             msg_W1[i][NODE:NODE + EDGE], msg_W2[i], r1(msg_b2[i]))
        agg = _sc_scatter(msg2, dst2, zer)
        nxt = i + 1
        with_pre = nxt < L
        wi = nxt if with_pre else i
        outs = _node_call(with_pre, h2, agg, mf2, cond2,
                          upd_W1[i][:NODE], upd_W1[i][NODE:NODE + HID],
                          upd_W1[i][NODE + HID:], r1(upd_b1[i]), upd_W2[i],
                          r1(upd_b2[i]), r1(ln_g[i]), r1(ln_b[i]),
                          msg_W1[wi][:NODE], msg_W1[wi][NODE + EDGE:], r1(msg_b1[wi]))
        if with_pre:
            h2, pre = outs
        else:
            (h2,) = outs

    return _final_call(h2.reshape(B, N, NODE), mf3, pool_W1, r1(pool_b1),
                       pool_W2, r1(pool_b2), head_W, r1(head_b))
